# baseline (device time: 8649 ns/iter reference)
import jax
import jax.numpy as jnp
from jax import lax
from jax.experimental import pallas as pl
from jax.experimental.pallas import tpu as pltpu

EPS = 1e-5
Y_SIZE = 2
BM = 128


def kernel(x, gamma):
    m, n = x.shape
    n_global = Y_SIZE * n
    nchunks = m // BM
    assert m % BM == 0

    def body(x_hbm, g_ref, out_hbm, x_vmem, o_vmem, comm_ref,
             in_sems, out_sems, send_sem, recv_sem):
        my_x = lax.axis_index("x")
        my_y = lax.axis_index("y")
        peer = (my_x, 1 - my_y)

        copies = []
        for i in range(nchunks):
            cp = pltpu.make_async_copy(
                x_hbm.at[pl.ds(i * BM, BM), :],
                x_vmem.at[pl.ds(i * BM, BM), :],
                in_sems.at[i],
            )
            cp.start()
            copies.append(cp)

        barrier_sem = pltpu.get_barrier_semaphore()
        pl.semaphore_signal(
            barrier_sem, inc=1, device_id=peer,
            device_id_type=pl.DeviceIdType.MESH,
        )
        pl.semaphore_wait(barrier_sem, 1)

        for i in range(nchunks):
            copies[i].wait()
            xc = x_vmem[pl.ds(i * BM, BM), :]
            comm_ref[0, :, pl.ds(i * BM, BM)] = jnp.sum(
                xc * xc, axis=1, keepdims=True).reshape(1, BM)

        rdma = pltpu.make_async_remote_copy(
            src_ref=comm_ref.at[0],
            dst_ref=comm_ref.at[1],
            send_sem=send_sem,
            recv_sem=recv_sem,
            device_id=peer,
            device_id_type=pl.DeviceIdType.MESH,
        )
        rdma.start()

        g = g_ref[:, :]
        for i in range(nchunks):
            sl = pl.ds(i * BM, BM)
            o_vmem[sl, :] = g * x_vmem[sl, :]

        rdma.wait()

        total = comm_ref[0, :, :] + comm_ref[1, :, :]
        inv = lax.rsqrt(total * (1.0 / n_global) + EPS).reshape(m, 1)

        outs = []
        for i in range(nchunks):
            sl = pl.ds(i * BM, BM)
            o_vmem[sl, :] = o_vmem[sl, :] * inv[i * BM:(i + 1) * BM, :]
            cp = pltpu.make_async_copy(
                o_vmem.at[sl, :], out_hbm.at[sl, :], out_sems.at[i])
            cp.start()
            outs.append(cp)
        for cp in outs:
            cp.wait()

    return pl.pallas_call(
        body,
        out_shape=jax.ShapeDtypeStruct((m, n), x.dtype),
        in_specs=[
            pl.BlockSpec(memory_space=pl.ANY),
            pl.BlockSpec(memory_space=pltpu.VMEM),
        ],
        out_specs=pl.BlockSpec(memory_space=pl.ANY),
        scratch_shapes=[
            pltpu.VMEM((m, n), x.dtype),
            pltpu.VMEM((m, n), x.dtype),
            pltpu.VMEM((2, 1, m), x.dtype),
            pltpu.SemaphoreType.DMA((m // BM,)),
            pltpu.SemaphoreType.DMA((m // BM,)),
            pltpu.SemaphoreType.DMA,
            pltpu.SemaphoreType.DMA,
        ],
        compiler_params=pltpu.CompilerParams(collective_id=0),
    )(x, gamma.reshape(1, n))


# device time: 7629 ns/iter; 1.1337x vs baseline; 1.1337x over previous
import jax
import jax.numpy as jnp
from jax import lax
from jax.experimental import pallas as pl
from jax.experimental.pallas import tpu as pltpu

EPS = 1e-5
Y_SIZE = 2
BM = 128


def kernel(x, gamma):
    m, n = x.shape
    n_global = Y_SIZE * n
    nchunks = m // BM
    assert m % BM == 0

    def body(x_hbm, g_ref, out_hbm, x_vmem, o_vmem, comm_ref,
             in_sems, out_sems, send_sems, recv_sems):
        my_x = lax.axis_index("x")
        my_y = lax.axis_index("y")
        peer = (my_x, 1 - my_y)

        copies = []
        for i in range(nchunks):
            sl = pl.ds(i * BM, BM)
            cp = pltpu.make_async_copy(
                x_hbm.at[sl, :], x_vmem.at[sl, :], in_sems.at[i])
            cp.start()
            copies.append(cp)

        barrier_sem = pltpu.get_barrier_semaphore()
        pl.semaphore_signal(
            barrier_sem, inc=1, device_id=peer,
            device_id_type=pl.DeviceIdType.MESH,
        )
        pl.semaphore_wait(barrier_sem, 1)

        rdmas = []
        for i in range(nchunks):
            sl = pl.ds(i * BM, BM)
            copies[i].wait()
            xc = x_vmem[sl, :]
            comm_ref[0, :, sl] = jnp.sum(
                xc * xc, axis=1, keepdims=True).reshape(1, BM)
            r = pltpu.make_async_remote_copy(
                src_ref=comm_ref.at[0, :, sl],
                dst_ref=comm_ref.at[1, :, sl],
                send_sem=send_sems.at[i],
                recv_sem=recv_sems.at[i],
                device_id=peer,
                device_id_type=pl.DeviceIdType.MESH,
            )
            r.start()
            rdmas.append(r)

        g = g_ref[:, :]
        outs = []
        for i in range(nchunks):
            sl = pl.ds(i * BM, BM)
            rdmas[i].wait_recv()
            tot = comm_ref[0, :, sl] + comm_ref[1, :, sl]
            inv = lax.rsqrt(tot * (1.0 / n_global) + EPS).reshape(BM, 1)
            o_vmem[sl, :] = g * x_vmem[sl, :] * inv
            cp = pltpu.make_async_copy(
                o_vmem.at[sl, :], out_hbm.at[sl, :], out_sems.at[i])
            cp.start()
            outs.append(cp)

        for r in rdmas:
            r.wait_send()
        for cp in outs:
            cp.wait()

    return pl.pallas_call(
        body,
        out_shape=jax.ShapeDtypeStruct((m, n), x.dtype),
        in_specs=[
            pl.BlockSpec(memory_space=pl.ANY),
            pl.BlockSpec(memory_space=pltpu.VMEM),
        ],
        out_specs=pl.BlockSpec(memory_space=pl.ANY),
        scratch_shapes=[
            pltpu.VMEM((m, n), x.dtype),
            pltpu.VMEM((m, n), x.dtype),
            pltpu.VMEM((2, 1, m), x.dtype),
            pltpu.SemaphoreType.DMA((m // BM,)),
            pltpu.SemaphoreType.DMA((m // BM,)),
            pltpu.SemaphoreType.DMA((m // BM,)),
            pltpu.SemaphoreType.DMA((m // BM,)),
        ],
        compiler_params=pltpu.CompilerParams(collective_id=0),
    )(x, gamma.reshape(1, n))


# device time: 7604 ns/iter; 1.1374x vs baseline; 1.0033x over previous
import jax
import jax.numpy as jnp
from jax import lax
from jax.experimental import pallas as pl
from jax.experimental.pallas import tpu as pltpu

EPS = 1e-5
Y_SIZE = 2


def kernel(x, gamma):
    m, n = x.shape
    n_global = Y_SIZE * n
    h = m // 2

    def body(x_ref, g_ref, out_ref, comm_ref, send_sems, recv_sems):
        my_x = lax.axis_index("x")
        my_y = lax.axis_index("y")
        peer = (my_x, 1 - my_y)

        barrier_sem = pltpu.get_barrier_semaphore()
        pl.semaphore_signal(
            barrier_sem, inc=1, device_id=peer,
            device_id_type=pl.DeviceIdType.MESH,
        )

        xc0 = x_ref[0:h, :]
        comm_ref[0, :, 0:h] = jnp.sum(
            xc0 * xc0, axis=1, keepdims=True).reshape(1, h)

        pl.semaphore_wait(barrier_sem, 1)

        rdmas = []
        for i in range(2):
            sl = pl.ds(i * h, h)
            if i == 1:
                xc = x_ref[sl, :]
                comm_ref[0, :, sl] = jnp.sum(
                    xc * xc, axis=1, keepdims=True).reshape(1, h)
            r = pltpu.make_async_remote_copy(
                src_ref=comm_ref.at[0, :, sl],
                dst_ref=comm_ref.at[1, :, sl],
                send_sem=send_sems.at[i],
                recv_sem=recv_sems.at[i],
                device_id=peer,
                device_id_type=pl.DeviceIdType.MESH,
            )
            r.start()
            rdmas.append(r)

        g = g_ref[:, :]
        scaled = g * x_ref[:, :]

        for i in range(2):
            sl = pl.ds(i * h, h)
            rdmas[i].wait_recv()
            tot = comm_ref[0, :, sl] + comm_ref[1, :, sl]
            inv = lax.rsqrt(tot * (1.0 / n_global) + EPS).reshape(h, 1)
            out_ref[sl, :] = scaled[i * h:(i + 1) * h, :] * inv

        for r in rdmas:
            r.wait_send()

    return pl.pallas_call(
        body,
        out_shape=jax.ShapeDtypeStruct((m, n), x.dtype),
        in_specs=[
            pl.BlockSpec(memory_space=pltpu.VMEM),
            pl.BlockSpec(memory_space=pltpu.VMEM),
        ],
        out_specs=pl.BlockSpec(memory_space=pltpu.VMEM),
        scratch_shapes=[
            pltpu.VMEM((2, 1, m), x.dtype),
            pltpu.SemaphoreType.DMA((2,)),
            pltpu.SemaphoreType.DMA((2,)),
        ],
        compiler_params=pltpu.CompilerParams(collective_id=0),
    )(x, gamma.reshape(1, n))


# device time: 6916 ns/iter; 1.2506x vs baseline; 1.0995x over previous
import jax
import jax.numpy as jnp
from jax import lax
from jax.experimental import pallas as pl
from jax.experimental.pallas import tpu as pltpu

EPS = 1e-5
Y_SIZE = 2


def kernel(x, gamma):
    m, n = x.shape
    n_global = Y_SIZE * n
    h = m // 2

    def body(x_ref, g_ref, out_ref, comm_ref, send_sems, recv_sems):
        my_x = lax.axis_index("x")
        my_y = lax.axis_index("y")
        peer = (my_x, 1 - my_y)

        barrier_sem = pltpu.get_barrier_semaphore()
        pl.semaphore_signal(
            barrier_sem, inc=1, device_id=peer,
            device_id_type=pl.DeviceIdType.MESH,
        )

        xc0 = x_ref[0:h, :]
        comm_ref[0, :, 0:h] = jnp.sum(
            xc0 * xc0, axis=1, keepdims=True).reshape(1, h)

        pl.semaphore_wait(barrier_sem, 1)

        rdmas = []
        for i in range(2):
            sl = pl.ds(i * h, h)
            if i == 1:
                xc = x_ref[sl, :]
                comm_ref[0, :, sl] = jnp.sum(
                    xc * xc, axis=1, keepdims=True).reshape(1, h)
            r = pltpu.make_async_remote_copy(
                src_ref=comm_ref.at[0, :, sl],
                dst_ref=comm_ref.at[1, :, sl],
                send_sem=send_sems.at[i],
                recv_sem=recv_sems.at[i],
                device_id=peer,
                device_id_type=pl.DeviceIdType.MESH,
            )
            r.start()
            rdmas.append(r)

        g = g_ref[:, :]
        scaled = g * x_ref[:, :]

        for i in range(2):
            sl = pl.ds(i * h, h)
            rdmas[i].wait_recv()
            tot = comm_ref[0, :, sl] + comm_ref[1, :, sl]
            inv = lax.rsqrt(tot * (1.0 / n_global) + EPS).reshape(h, 1)
            out_ref[sl, :] = scaled[i * h:(i + 1) * h, :] * inv

        for r in rdmas:
            r.wait_send()

    return pl.pallas_call(
        body,
        out_shape=jax.ShapeDtypeStruct((m, n), x.dtype),
        in_specs=[
            pl.BlockSpec(memory_space=pltpu.VMEM),
            pl.BlockSpec(memory_space=pltpu.VMEM),
        ],
        out_specs=pl.BlockSpec(memory_space=pltpu.VMEM),
        scratch_shapes=[
            pltpu.VMEM((2, 1, m), x.dtype),
            pltpu.SemaphoreType.DMA((2,)),
            pltpu.SemaphoreType.DMA((2,)),
        ],
        compiler_params=pltpu.CompilerParams(collective_id=0),
        input_output_aliases={0: 0},
    )(x, gamma.reshape(1, n))
